# initial kernel scaffold (unmeasured)
import jax
import jax.numpy as jnp
from jax import lax
from jax.experimental import pallas as pl
from jax.experimental.pallas import tpu as pltpu

N_DEV = 4
SQ = 2048
HQ = 8
DH = 128
DM = HQ * DH
HALO = 128
GLOB = 32
GSLOT = 128
OWN = GSLOT + HALO
KBUF = GSLOT + HALO + SQ + HALO
QBLK = 512
SCALE = 0.08838834764831843
NEG = -1e9


def kernel(x, Wq, K_ext, V_ext, Wo):
    def body(x_ref, wq_ref, k_ref, v_ref, wo_ref, out_ref,
             qbuf, kbuf, vbuf, qg, ctx,
             po, pm, plb, rxo, rxm, rxl,
             halo_send, halo_recv, gsend, grecv, psend, precv):
        my = lax.axis_index("i")
        left = lax.rem(my + N_DEV - 1, N_DEV)
        right = lax.rem(my + 1, N_DEV)

        bsem = pltpu.get_barrier_semaphore()
        for nbr in (left, right):
            pl.semaphore_signal(bsem, inc=1, device_id=(nbr,),
                                device_id_type=pl.DeviceIdType.MESH)
        on_diag = jnp.logical_or(my == 0, my == 2)

        @pl.when(on_diag)
        def _():
            other = jnp.where(my == 0, 2, 0)
            pl.semaphore_signal(bsem, inc=1, device_id=(other,),
                                device_id_type=pl.DeviceIdType.MESH)
            pl.semaphore_wait(bsem, 3)

        @pl.when(jnp.logical_not(on_diag))
        def _():
            pl.semaphore_wait(bsem, 2)

        xb = x_ref[0].astype(jnp.bfloat16)
        wq = wq_ref[...].astype(jnp.bfloat16)
        qbuf[...] = lax.dot(xb, wq,
                            preferred_element_type=jnp.float32).astype(jnp.bfloat16)
        kbuf[OWN:OWN + SQ, :] = k_ref[0].reshape(SQ, DM).astype(jnp.bfloat16)
        vbuf[OWN:OWN + SQ, :] = v_ref[0].reshape(SQ, DM).astype(jnp.bfloat16)
        kbuf[GLOB:GSLOT, :] = jnp.zeros((GSLOT - GLOB, DM), jnp.bfloat16)
        vbuf[GLOB:GSLOT, :] = jnp.zeros((GSLOT - GLOB, DM), jnp.bfloat16)

        @pl.when(my == 0)
        def _():
            kbuf[0:GLOB, :] = kbuf[OWN:OWN + GLOB, :]
            vbuf[0:GLOB, :] = vbuf[OWN:OWN + GLOB, :]
            qg[...] = qbuf[0:GLOB, :]

        halo_rdmas = []
        for buf, s0 in ((kbuf, 0), (vbuf, 2)):
            halo_rdmas.append(pltpu.make_async_remote_copy(
                src_ref=buf.at[pl.ds(OWN, HALO)],
                dst_ref=buf.at[pl.ds(OWN + SQ, HALO)],
                send_sem=halo_send.at[s0], recv_sem=halo_recv.at[s0],
                device_id=(left,), device_id_type=pl.DeviceIdType.MESH))
            halo_rdmas.append(pltpu.make_async_remote_copy(
                src_ref=buf.at[pl.ds(OWN + SQ - HALO, HALO)],
                dst_ref=buf.at[pl.ds(GSLOT, HALO)],
                send_sem=halo_send.at[s0 + 1], recv_sem=halo_recv.at[s0 + 1],
                device_id=(right,), device_id_type=pl.DeviceIdType.MESH))
        for r in halo_rdmas:
            r.start()

        @pl.when(my == 0)
        def _():
            sends = []
            i = 0
            for dst in (1, 2, 3):
                for src_r, dst_r, j in (
                        (kbuf.at[pl.ds(OWN, GLOB)], kbuf.at[pl.ds(0, GLOB)], 0),
                        (vbuf.at[pl.ds(OWN, GLOB)], vbuf.at[pl.ds(0, GLOB)], 1),
                        (qbuf.at[pl.ds(0, GLOB)], qg, 2)):
                    d = pltpu.make_async_remote_copy(
                        src_ref=src_r, dst_ref=dst_r,
                        send_sem=gsend.at[i], recv_sem=grecv.at[j],
                        device_id=(dst,), device_id_type=pl.DeviceIdType.MESH)
                    d.start()
                    sends.append(d)
                    i += 1
            for d in sends:
                d.wait_send()

        def recv_only(dst_r, rsem):
            return pltpu.make_async_remote_copy(
                src_ref=dst_r, dst_ref=dst_r, send_sem=gsend.at[0],
                recv_sem=rsem, device_id=(0,),
                device_id_type=pl.DeviceIdType.MESH)

        @pl.when(my != 0)
        def _():
            recv_only(qg, grecv.at[2]).wait_recv()

        qgb = qg[...]
        for h in range(HQ):
            qh = qgb[:, h * DH:(h + 1) * DH]
            s = lax.dot_general(qh, kbuf[OWN:OWN + SQ, h * DH:(h + 1) * DH],
                                (((1,), (1,)), ((), ())),
                                preferred_element_type=jnp.float32) * SCALE
            m = jnp.max(s, axis=1, keepdims=True)
            w = jnp.exp(s - m)
            l = jnp.sum(w, axis=1, keepdims=True)
            o = lax.dot_general(w.astype(jnp.bfloat16),
                                vbuf[OWN:OWN + SQ, h * DH:(h + 1) * DH],
                                (((1,), (0,)), ((), ())),
                                preferred_element_type=jnp.float32)
            po[h * GLOB:(h + 1) * GLOB, :] = o
            pm[h * GLOB:(h + 1) * GLOB, :] = jnp.broadcast_to(m, (GLOB, DH))
            plb[h * GLOB:(h + 1) * GLOB, :] = jnp.broadcast_to(l, (GLOB, DH))

        for src in (1, 2, 3):
            @pl.when(my == src)
            def _(src=src):
                ds = []
                for j, (sbuf, rbuf) in enumerate(
                        ((po, rxo), (pm, rxm), (plb, rxl))):
                    d = pltpu.make_async_remote_copy(
                        src_ref=sbuf, dst_ref=rbuf.at[src - 1],
                        send_sem=psend.at[j], recv_sem=precv.at[src - 1, j],
                        device_id=(0,), device_id_type=pl.DeviceIdType.MESH)
                    d.start()
                    ds.append(d)
                for d in ds:
                    d.wait_send()

        for r in halo_rdmas:
            r.wait()

        @pl.when(my != 0)
        def _():
            recv_only(kbuf.at[pl.ds(0, GLOB)], grecv.at[0]).wait_recv()
            recv_only(vbuf.at[pl.ds(0, GLOB)], grecv.at[1]).wait_recv()

        def qb_body(qb, carry):
            q0 = qb * QBLK
            ci = lax.broadcasted_iota(jnp.int32, (QBLK, KBUF), 1)
            qi = my * SQ + q0 + lax.broadcasted_iota(jnp.int32, (QBLK, KBUF), 0)
            ki = my * SQ + ci - OWN
            band = (jnp.abs(qi - ki) <= HALO) & (ki >= GLOB) & (ci >= GSLOT)
            band = band & jnp.logical_not(
                jnp.logical_and(my == N_DEV - 1, ci >= OWN + SQ))
            mask = band | (ci < GLOB)
            qrows = qbuf[pl.ds(q0, QBLK), :]
            for h in range(HQ):
                qh = qrows[:, h * DH:(h + 1) * DH]
                s = lax.dot_general(qh, kbuf[:, h * DH:(h + 1) * DH],
                                    (((1,), (1,)), ((), ())),
                                    preferred_element_type=jnp.float32) * SCALE
                s = jnp.where(mask, s, NEG)
                m = jnp.max(s, axis=1, keepdims=True)
                w = jnp.exp(s - m)
                l = jnp.sum(w, axis=1, keepdims=True)
                o = lax.dot_general(w.astype(jnp.bfloat16),
                                    vbuf[:, h * DH:(h + 1) * DH],
                                    (((1,), (0,)), ((), ())),
                                    preferred_element_type=jnp.float32)
                ctx[pl.ds(q0, QBLK), h * DH:(h + 1) * DH] = \
                    (o / l).astype(jnp.bfloat16)
            return carry

        lax.fori_loop(0, SQ // QBLK, qb_body, 0)

        @pl.when(my == 0)
        def _():
            for s in range(3):
                for j, rbuf in enumerate((rxo, rxm, rxl)):
                    pltpu.make_async_remote_copy(
                        src_ref=rbuf.at[s], dst_ref=rbuf.at[s],
                        send_sem=psend.at[0], recv_sem=precv.at[s, j],
                        device_id=(0,),
                        device_id_type=pl.DeviceIdType.MESH).wait_recv()
            m0 = pm[...]
            m1, m2, m3 = rxm[0], rxm[1], rxm[2]
            mx = jnp.maximum(jnp.maximum(m0, m1), jnp.maximum(m2, m3))
            e0 = jnp.exp(m0 - mx)
            e1 = jnp.exp(m1 - mx)
            e2 = jnp.exp(m2 - mx)
            e3 = jnp.exp(m3 - mx)
            osum = po[...] * e0 + rxo[0] * e1 + rxo[1] * e2 + rxo[2] * e3
            lsum = plb[...] * e0 + rxl[0] * e1 + rxl[1] * e2 + rxl[2] * e3
            cg = (osum / lsum).astype(jnp.bfloat16)
            for h in range(HQ):
                ctx[0:GLOB, h * DH:(h + 1) * DH] = cg[h * GLOB:(h + 1) * GLOB, :]

        out_ref[...] = lax.dot(ctx[...], wo_ref[...].astype(jnp.bfloat16),
                               preferred_element_type=jnp.float32)[None]

    return pl.pallas_call(
        body,
        out_shape=jax.ShapeDtypeStruct((1, SQ, DM), jnp.float32),
        in_specs=[pl.BlockSpec(memory_space=pltpu.VMEM)] * 5,
        out_specs=pl.BlockSpec(memory_space=pltpu.VMEM),
        scratch_shapes=[
            pltpu.VMEM((SQ, DM), jnp.bfloat16),
            pltpu.VMEM((KBUF, DM), jnp.bfloat16),
            pltpu.VMEM((KBUF, DM), jnp.bfloat16),
            pltpu.VMEM((GLOB, DM), jnp.bfloat16),
            pltpu.VMEM((SQ, DM), jnp.bfloat16),
            pltpu.VMEM((HQ * GLOB, DH), jnp.float32),
            pltpu.VMEM((HQ * GLOB, DH), jnp.float32),
            pltpu.VMEM((HQ * GLOB, DH), jnp.float32),
            pltpu.VMEM((3, HQ * GLOB, DH), jnp.float32),
            pltpu.VMEM((3, HQ * GLOB, DH), jnp.float32),
            pltpu.VMEM((3, HQ * GLOB, DH), jnp.float32),
            pltpu.SemaphoreType.DMA((4,)),
            pltpu.SemaphoreType.DMA((4,)),
            pltpu.SemaphoreType.DMA((9,)),
            pltpu.SemaphoreType.DMA((3,)),
            pltpu.SemaphoreType.DMA((3,)),
            pltpu.SemaphoreType.DMA((3, 3)),
        ],
        compiler_params=pltpu.CompilerParams(collective_id=0),
    )(x, Wq, K_ext, V_ext, Wo)


# baseline (device time: 173542 ns/iter reference)
import jax
import jax.numpy as jnp
from jax import lax
from jax.experimental import pallas as pl
from jax.experimental.pallas import tpu as pltpu

N_DEV = 4
SQ = 2048
HQ = 8
DH = 128
DM = HQ * DH
HALO = 128
GLOB = 32
GSLOT = 128
OWN = GSLOT + HALO
KBUF = GSLOT + HALO + SQ + HALO
QBLK = 256
CHUNK = 512
SCALE = 0.08838834764831843
NEG = -1e9


def kernel(x, Wq, K_ext, V_ext, Wo):
    def body(x_ref, wq_ref, k_ref, v_ref, wo_ref, out_ref,
             qbuf, kbuf, vbuf, qg, ctx, xstage, kvstage,
             po, pm, plb, rxo, rxm, rxl,
             lsem, halo_send, halo_recv, gsend, grecv, psend, precv):
        my = lax.axis_index("i")
        left = lax.rem(my + N_DEV - 1, N_DEV)
        right = lax.rem(my + 1, N_DEV)

        bsem = pltpu.get_barrier_semaphore()
        for nbr in (left, right):
            pl.semaphore_signal(bsem, inc=1, device_id=(nbr,),
                                device_id_type=pl.DeviceIdType.MESH)
        on_diag = jnp.logical_or(my == 0, my == 2)

        @pl.when(on_diag)
        def _():
            other = jnp.where(my == 0, 2, 0)
            pl.semaphore_signal(bsem, inc=1, device_id=(other,),
                                device_id_type=pl.DeviceIdType.MESH)
            pl.semaphore_wait(bsem, 3)

        @pl.when(jnp.logical_not(on_diag))
        def _():
            pl.semaphore_wait(bsem, 2)

        wqb = wq_ref[...].astype(jnp.bfloat16)
        for c in range(SQ // CHUNK):
            cp = pltpu.make_async_copy(
                x_ref.at[0, pl.ds(c * CHUNK, CHUNK)], xstage, lsem)
            cp.start()
            cp.wait()
            qbuf[pl.ds(c * CHUNK, CHUNK), :] = lax.dot(
                xstage[...].astype(jnp.bfloat16), wqb,
                preferred_element_type=jnp.float32).astype(jnp.bfloat16)
        for src_r, dst in ((k_ref, kbuf), (v_ref, vbuf)):
            for c in range(SQ // CHUNK):
                cp = pltpu.make_async_copy(
                    src_r.at[0, pl.ds(c * CHUNK, CHUNK)], kvstage, lsem)
                cp.start()
                cp.wait()
                dst[pl.ds(OWN + c * CHUNK, CHUNK), :] = \
                    kvstage[...].reshape(CHUNK, DM).astype(jnp.bfloat16)
        kbuf[GLOB:GSLOT, :] = jnp.zeros((GSLOT - GLOB, DM), jnp.bfloat16)
        vbuf[GLOB:GSLOT, :] = jnp.zeros((GSLOT - GLOB, DM), jnp.bfloat16)

        @pl.when(my == 0)
        def _():
            kbuf[0:GLOB, :] = kbuf[OWN:OWN + GLOB, :]
            vbuf[0:GLOB, :] = vbuf[OWN:OWN + GLOB, :]
            qg[...] = qbuf[0:GLOB, :]

        halo_rdmas = []
        for buf, s0 in ((kbuf, 0), (vbuf, 2)):
            halo_rdmas.append(pltpu.make_async_remote_copy(
                src_ref=buf.at[pl.ds(OWN, HALO)],
                dst_ref=buf.at[pl.ds(OWN + SQ, HALO)],
                send_sem=halo_send.at[s0], recv_sem=halo_recv.at[s0],
                device_id=(left,), device_id_type=pl.DeviceIdType.MESH))
            halo_rdmas.append(pltpu.make_async_remote_copy(
                src_ref=buf.at[pl.ds(OWN + SQ - HALO, HALO)],
                dst_ref=buf.at[pl.ds(GSLOT, HALO)],
                send_sem=halo_send.at[s0 + 1], recv_sem=halo_recv.at[s0 + 1],
                device_id=(right,), device_id_type=pl.DeviceIdType.MESH))
        for r in halo_rdmas:
            r.start()

        @pl.when(my == 0)
        def _():
            sends = []
            i = 0
            for dst in (1, 2, 3):
                for src_r, dst_r, j in (
                        (kbuf.at[pl.ds(OWN, GLOB)], kbuf.at[pl.ds(0, GLOB)], 0),
                        (vbuf.at[pl.ds(OWN, GLOB)], vbuf.at[pl.ds(0, GLOB)], 1),
                        (qbuf.at[pl.ds(0, GLOB)], qg, 2)):
                    d = pltpu.make_async_remote_copy(
                        src_ref=src_r, dst_ref=dst_r,
                        send_sem=gsend.at[i], recv_sem=grecv.at[j],
                        device_id=(dst,), device_id_type=pl.DeviceIdType.MESH)
                    d.start()
                    sends.append(d)
                    i += 1
            for d in sends:
                d.wait_send()

        def recv_only(dst_r, rsem):
            return pltpu.make_async_remote_copy(
                src_ref=dst_r, dst_ref=dst_r, send_sem=gsend.at[0],
                recv_sem=rsem, device_id=(0,),
                device_id_type=pl.DeviceIdType.MESH)

        @pl.when(my != 0)
        def _():
            recv_only(qg, grecv.at[2]).wait_recv()

        qgb = qg[...]
        for h in range(HQ):
            qh = qgb[:, h * DH:(h + 1) * DH]
            s = lax.dot_general(qh, kbuf[OWN:OWN + SQ, h * DH:(h + 1) * DH],
                                (((1,), (1,)), ((), ())),
                                preferred_element_type=jnp.float32) * SCALE
            m = jnp.max(s, axis=1, keepdims=True)
            w = jnp.exp(s - m)
            l = jnp.sum(w, axis=1, keepdims=True)
            o = lax.dot_general(w.astype(jnp.bfloat16),
                                vbuf[OWN:OWN + SQ, h * DH:(h + 1) * DH],
                                (((1,), (0,)), ((), ())),
                                preferred_element_type=jnp.float32)
            po[h * GLOB:(h + 1) * GLOB, :] = o
            pm[h * GLOB:(h + 1) * GLOB, :] = jnp.broadcast_to(m, (GLOB, DH))
            plb[h * GLOB:(h + 1) * GLOB, :] = jnp.broadcast_to(l, (GLOB, DH))

        for src in (1, 2, 3):
            @pl.when(my == src)
            def _(src=src):
                ds = []
                for j, (sbuf, rbuf) in enumerate(
                        ((po, rxo), (pm, rxm), (plb, rxl))):
                    d = pltpu.make_async_remote_copy(
                        src_ref=sbuf, dst_ref=rbuf.at[src - 1],
                        send_sem=psend.at[j], recv_sem=precv.at[src - 1, j],
                        device_id=(0,), device_id_type=pl.DeviceIdType.MESH)
                    d.start()
                    ds.append(d)
                for d in ds:
                    d.wait_send()

        for r in halo_rdmas:
            r.wait()

        @pl.when(my != 0)
        def _():
            recv_only(kbuf.at[pl.ds(0, GLOB)], grecv.at[0]).wait_recv()
            recv_only(vbuf.at[pl.ds(0, GLOB)], grecv.at[1]).wait_recv()

        ci = lax.broadcasted_iota(jnp.int32, (1, KBUF), 1)
        ki = my * SQ + ci - OWN
        kvalid = (ki >= GLOB) & (ci >= GSLOT)
        kvalid = kvalid & jnp.logical_not(
            jnp.logical_and(my == N_DEV - 1, ci >= OWN + SQ))
        kglob = ci < GLOB

        def qb_body(qb, carry):
            q0 = qb * QBLK
            qi = my * SQ + q0 + lax.broadcasted_iota(jnp.int32, (QBLK, 1), 0)
            band = (ki >= qi - HALO) & (ki <= qi + HALO) & kvalid
            bias = jnp.where(band | kglob, 0.0, NEG).astype(jnp.float32)
            def h_body(h, hc):
                c0 = h * DH
                qh = qbuf[pl.ds(q0, QBLK), pl.ds(c0, DH)]
                s = lax.dot_general(qh, kbuf[:, pl.ds(c0, DH)],
                                    (((1,), (1,)), ((), ())),
                                    preferred_element_type=jnp.float32)
                s = s * SCALE + bias
                m = jnp.max(s, axis=1, keepdims=True)
                w = jnp.exp(s - m)
                l = jnp.sum(w, axis=1, keepdims=True)
                o = lax.dot_general(w.astype(jnp.bfloat16),
                                    vbuf[:, pl.ds(c0, DH)],
                                    (((1,), (0,)), ((), ())),
                                    preferred_element_type=jnp.float32)
                ctx[pl.ds(q0, QBLK), pl.ds(c0, DH)] = \
                    (o / l).astype(jnp.bfloat16)
                return hc

            lax.fori_loop(0, HQ, h_body, 0)
            return carry

        lax.fori_loop(0, SQ // QBLK, qb_body, 0)

        @pl.when(my == 0)
        def _():
            for s in range(3):
                for j, rbuf in enumerate((rxo, rxm, rxl)):
                    pltpu.make_async_remote_copy(
                        src_ref=rbuf.at[s], dst_ref=rbuf.at[s],
                        send_sem=psend.at[0], recv_sem=precv.at[s, j],
                        device_id=(0,),
                        device_id_type=pl.DeviceIdType.MESH).wait_recv()
            m0 = pm[...]
            m1, m2, m3 = rxm[0], rxm[1], rxm[2]
            mx = jnp.maximum(jnp.maximum(m0, m1), jnp.maximum(m2, m3))
            e0 = jnp.exp(m0 - mx)
            e1 = jnp.exp(m1 - mx)
            e2 = jnp.exp(m2 - mx)
            e3 = jnp.exp(m3 - mx)
            osum = po[...] * e0 + rxo[0] * e1 + rxo[1] * e2 + rxo[2] * e3
            lsum = plb[...] * e0 + rxl[0] * e1 + rxl[1] * e2 + rxl[2] * e3
            cg = (osum / lsum).astype(jnp.bfloat16)
            for h in range(HQ):
                ctx[0:GLOB, h * DH:(h + 1) * DH] = cg[h * GLOB:(h + 1) * GLOB, :]

        wob = wo_ref[...].astype(jnp.bfloat16)
        for c in range(SQ // CHUNK):
            out_ref[0, pl.ds(c * CHUNK, CHUNK), :] = lax.dot(
                ctx[pl.ds(c * CHUNK, CHUNK), :], wob,
                preferred_element_type=jnp.float32)

    return pl.pallas_call(
        body,
        out_shape=jax.ShapeDtypeStruct((1, SQ, DM), jnp.float32),
        in_specs=[
            pl.BlockSpec(memory_space=pl.ANY),
            pl.BlockSpec(memory_space=pltpu.VMEM),
            pl.BlockSpec(memory_space=pl.ANY),
            pl.BlockSpec(memory_space=pl.ANY),
            pl.BlockSpec(memory_space=pltpu.VMEM),
        ],
        out_specs=pl.BlockSpec(memory_space=pltpu.VMEM),
        scratch_shapes=[
            pltpu.VMEM((SQ, DM), jnp.bfloat16),
            pltpu.VMEM((KBUF, DM), jnp.bfloat16),
            pltpu.VMEM((KBUF, DM), jnp.bfloat16),
            pltpu.VMEM((GLOB, DM), jnp.bfloat16),
            pltpu.VMEM((SQ, DM), jnp.bfloat16),
            pltpu.VMEM((CHUNK, DM), jnp.float32),
            pltpu.VMEM((CHUNK, HQ, DH), jnp.float32),
            pltpu.VMEM((HQ * GLOB, DH), jnp.float32),
            pltpu.VMEM((HQ * GLOB, DH), jnp.float32),
            pltpu.VMEM((HQ * GLOB, DH), jnp.float32),
            pltpu.VMEM((3, HQ * GLOB, DH), jnp.float32),
            pltpu.VMEM((3, HQ * GLOB, DH), jnp.float32),
            pltpu.VMEM((3, HQ * GLOB, DH), jnp.float32),
            pltpu.SemaphoreType.DMA,
            pltpu.SemaphoreType.DMA((4,)),
            pltpu.SemaphoreType.DMA((4,)),
            pltpu.SemaphoreType.DMA((9,)),
            pltpu.SemaphoreType.DMA((3,)),
            pltpu.SemaphoreType.DMA((3,)),
            pltpu.SemaphoreType.DMA((3, 3)),
        ],
        compiler_params=pltpu.CompilerParams(
            collective_id=0, vmem_limit_bytes=46 * 1024 * 1024),
    )(x, Wq, K_ext, V_ext, Wo)


# device time: 113987 ns/iter; 1.5225x vs baseline; 1.5225x over previous
import jax
import jax.numpy as jnp
from jax import lax
from jax.experimental import pallas as pl
from jax.experimental.pallas import tpu as pltpu

N_DEV = 4
SQ = 2048
HQ = 8
DH = 128
DM = HQ * DH
HALO = 128
GLOB = 32
GSLOT = 128
OWN = GSLOT + HALO
KBUF = GSLOT + HALO + SQ + HALO
QBLK = 256
CHUNK = 512
SCALE = 0.08838834764831843
NEG = -1e9


def kernel(x, Wq, K_ext, V_ext, Wo):
    def body(x_ref, wq_ref, k_ref, v_ref, wo_ref, out_ref,
             qbuf, kbuf, vbuf, qg, ctx, xstage, kvstage,
             po, pm, plb, rxo, rxm, rxl,
             lsem, halo_send, halo_recv, gsend, grecv, psend, precv):
        my = lax.axis_index("i")
        left = lax.rem(my + N_DEV - 1, N_DEV)
        right = lax.rem(my + 1, N_DEV)

        bsem = pltpu.get_barrier_semaphore()
        for nbr in (left, right):
            pl.semaphore_signal(bsem, inc=1, device_id=(nbr,),
                                device_id_type=pl.DeviceIdType.MESH)
        on_diag = jnp.logical_or(my == 0, my == 2)

        @pl.when(on_diag)
        def _():
            other = jnp.where(my == 0, 2, 0)
            pl.semaphore_signal(bsem, inc=1, device_id=(other,),
                                device_id_type=pl.DeviceIdType.MESH)
            pl.semaphore_wait(bsem, 3)

        @pl.when(jnp.logical_not(on_diag))
        def _():
            pl.semaphore_wait(bsem, 2)

        wqb = wq_ref[...].astype(jnp.bfloat16)
        for c in range(SQ // CHUNK):
            cp = pltpu.make_async_copy(
                x_ref.at[0, pl.ds(c * CHUNK, CHUNK)], xstage, lsem)
            cp.start()
            cp.wait()
            qbuf[pl.ds(c * CHUNK, CHUNK), :] = lax.dot(
                xstage[...].astype(jnp.bfloat16), wqb,
                preferred_element_type=jnp.float32).astype(jnp.bfloat16)
        for src_r, dst in ((k_ref, kbuf), (v_ref, vbuf)):
            for c in range(SQ // CHUNK):
                cp = pltpu.make_async_copy(
                    src_r.at[0, pl.ds(c * CHUNK, CHUNK)], kvstage, lsem)
                cp.start()
                cp.wait()
                dst[pl.ds(OWN + c * CHUNK, CHUNK), :] = \
                    kvstage[...].reshape(CHUNK, DM).astype(jnp.bfloat16)
        kbuf[GLOB:GSLOT, :] = jnp.zeros((GSLOT - GLOB, DM), jnp.bfloat16)
        vbuf[GLOB:GSLOT, :] = jnp.zeros((GSLOT - GLOB, DM), jnp.bfloat16)

        @pl.when(my == 0)
        def _():
            kbuf[0:GLOB, :] = kbuf[OWN:OWN + GLOB, :]
            vbuf[0:GLOB, :] = vbuf[OWN:OWN + GLOB, :]
            qg[...] = qbuf[0:GLOB, :]

        halo_rdmas = []
        for buf, s0 in ((kbuf, 0), (vbuf, 2)):
            halo_rdmas.append(pltpu.make_async_remote_copy(
                src_ref=buf.at[pl.ds(OWN, HALO)],
                dst_ref=buf.at[pl.ds(OWN + SQ, HALO)],
                send_sem=halo_send.at[s0], recv_sem=halo_recv.at[s0],
                device_id=(left,), device_id_type=pl.DeviceIdType.MESH))
            halo_rdmas.append(pltpu.make_async_remote_copy(
                src_ref=buf.at[pl.ds(OWN + SQ - HALO, HALO)],
                dst_ref=buf.at[pl.ds(GSLOT, HALO)],
                send_sem=halo_send.at[s0 + 1], recv_sem=halo_recv.at[s0 + 1],
                device_id=(right,), device_id_type=pl.DeviceIdType.MESH))
        for r in halo_rdmas:
            r.start()

        @pl.when(my == 0)
        def _():
            sends = []
            i = 0
            for dst in (1, 2, 3):
                for src_r, dst_r, j in (
                        (kbuf.at[pl.ds(OWN, GLOB)], kbuf.at[pl.ds(0, GLOB)], 0),
                        (vbuf.at[pl.ds(OWN, GLOB)], vbuf.at[pl.ds(0, GLOB)], 1),
                        (qbuf.at[pl.ds(0, GLOB)], qg, 2)):
                    d = pltpu.make_async_remote_copy(
                        src_ref=src_r, dst_ref=dst_r,
                        send_sem=gsend.at[i], recv_sem=grecv.at[j],
                        device_id=(dst,), device_id_type=pl.DeviceIdType.MESH)
                    d.start()
                    sends.append(d)
                    i += 1
            for d in sends:
                d.wait_send()

        def recv_only(dst_r, rsem):
            return pltpu.make_async_remote_copy(
                src_ref=dst_r, dst_ref=dst_r, send_sem=gsend.at[0],
                recv_sem=rsem, device_id=(0,),
                device_id_type=pl.DeviceIdType.MESH)

        @pl.when(my != 0)
        def _():
            recv_only(qg, grecv.at[2]).wait_recv()

        qgb = qg[...]
        for h in range(HQ):
            qh = qgb[:, h * DH:(h + 1) * DH]
            s = lax.dot_general(qh, kbuf[OWN:OWN + SQ, h * DH:(h + 1) * DH],
                                (((1,), (1,)), ((), ())),
                                preferred_element_type=jnp.float32) * SCALE
            m = jnp.max(s, axis=1, keepdims=True)
            w = jnp.exp(s - m)
            l = jnp.sum(w, axis=1, keepdims=True)
            o = lax.dot_general(w.astype(jnp.bfloat16),
                                vbuf[OWN:OWN + SQ, h * DH:(h + 1) * DH],
                                (((1,), (0,)), ((), ())),
                                preferred_element_type=jnp.float32)
            po[h * GLOB:(h + 1) * GLOB, :] = o
            pm[h * GLOB:(h + 1) * GLOB, :] = jnp.broadcast_to(m, (GLOB, DH))
            plb[h * GLOB:(h + 1) * GLOB, :] = jnp.broadcast_to(l, (GLOB, DH))

        for src in (1, 2, 3):
            @pl.when(my == src)
            def _(src=src):
                ds = []
                for j, (sbuf, rbuf) in enumerate(
                        ((po, rxo), (pm, rxm), (plb, rxl))):
                    d = pltpu.make_async_remote_copy(
                        src_ref=sbuf, dst_ref=rbuf.at[src - 1],
                        send_sem=psend.at[j], recv_sem=precv.at[src - 1, j],
                        device_id=(0,), device_id_type=pl.DeviceIdType.MESH)
                    d.start()
                    ds.append(d)
                for d in ds:
                    d.wait_send()

        for r in halo_rdmas:
            r.wait()

        @pl.when(my != 0)
        def _():
            recv_only(kbuf.at[pl.ds(0, GLOB)], grecv.at[0]).wait_recv()
            recv_only(vbuf.at[pl.ds(0, GLOB)], grecv.at[1]).wait_recv()

        WWIN = QBLK + 2 * HALO
        cg = lax.broadcasted_iota(jnp.int32, (1, GSLOT), 1)
        bias_g = jnp.where(cg < GLOB, 0.0, NEG).astype(jnp.float32)

        def qb_body(qb, carry):
            q0 = qb * QBLK
            cw = lax.broadcasted_iota(jnp.int32, (1, WWIN), 1)
            ciw = GSLOT + q0 + cw
            kiw = my * SQ + ciw - OWN
            qi = my * SQ + q0 + lax.broadcasted_iota(jnp.int32, (QBLK, 1), 0)
            band = (kiw >= qi - HALO) & (kiw <= qi + HALO) & (kiw >= GLOB)
            band = band & jnp.logical_not(
                jnp.logical_and(my == N_DEV - 1, ciw >= OWN + SQ))
            bias = jnp.where(band, 0.0, NEG).astype(jnp.float32)

            def h_body(h, hc):
                c0 = h * DH
                qh = qbuf[pl.ds(q0, QBLK), pl.ds(c0, DH)]
                sw = lax.dot_general(qh, kbuf[pl.ds(GSLOT + q0, WWIN),
                                              pl.ds(c0, DH)],
                                     (((1,), (1,)), ((), ())),
                                     preferred_element_type=jnp.float32)
                sw = sw * SCALE + bias
                sg = lax.dot_general(qh, kbuf[0:GSLOT, pl.ds(c0, DH)],
                                     (((1,), (1,)), ((), ())),
                                     preferred_element_type=jnp.float32)
                sg = sg * SCALE + bias_g
                m = jnp.maximum(jnp.max(sw, axis=1, keepdims=True),
                                jnp.max(sg, axis=1, keepdims=True))
                ww = jnp.exp(sw - m)
                wg = jnp.exp(sg - m)
                l = (jnp.sum(ww, axis=1, keepdims=True)
                     + jnp.sum(wg, axis=1, keepdims=True))
                o = lax.dot_general(ww.astype(jnp.bfloat16),
                                    vbuf[pl.ds(GSLOT + q0, WWIN),
                                         pl.ds(c0, DH)],
                                    (((1,), (0,)), ((), ())),
                                    preferred_element_type=jnp.float32)
                o = o + lax.dot_general(wg.astype(jnp.bfloat16),
                                        vbuf[0:GSLOT, pl.ds(c0, DH)],
                                        (((1,), (0,)), ((), ())),
                                        preferred_element_type=jnp.float32)
                ctx[pl.ds(q0, QBLK), pl.ds(c0, DH)] = \
                    (o / l).astype(jnp.bfloat16)
                return hc

            lax.fori_loop(0, HQ, h_body, 0)
            return carry

        lax.fori_loop(0, SQ // QBLK, qb_body, 0)

        @pl.when(my == 0)
        def _():
            for s in range(3):
                for j, rbuf in enumerate((rxo, rxm, rxl)):
                    pltpu.make_async_remote_copy(
                        src_ref=rbuf.at[s], dst_ref=rbuf.at[s],
                        send_sem=psend.at[0], recv_sem=precv.at[s, j],
                        device_id=(0,),
                        device_id_type=pl.DeviceIdType.MESH).wait_recv()
            m0 = pm[...]
            m1, m2, m3 = rxm[0], rxm[1], rxm[2]
            mx = jnp.maximum(jnp.maximum(m0, m1), jnp.maximum(m2, m3))
            e0 = jnp.exp(m0 - mx)
            e1 = jnp.exp(m1 - mx)
            e2 = jnp.exp(m2 - mx)
            e3 = jnp.exp(m3 - mx)
            osum = po[...] * e0 + rxo[0] * e1 + rxo[1] * e2 + rxo[2] * e3
            lsum = plb[...] * e0 + rxl[0] * e1 + rxl[1] * e2 + rxl[2] * e3
            cg = (osum / lsum).astype(jnp.bfloat16)
            for h in range(HQ):
                ctx[0:GLOB, h * DH:(h + 1) * DH] = cg[h * GLOB:(h + 1) * GLOB, :]

        wob = wo_ref[...].astype(jnp.bfloat16)
        for c in range(SQ // CHUNK):
            out_ref[0, pl.ds(c * CHUNK, CHUNK), :] = lax.dot(
                ctx[pl.ds(c * CHUNK, CHUNK), :], wob,
                preferred_element_type=jnp.float32)

    return pl.pallas_call(
        body,
        out_shape=jax.ShapeDtypeStruct((1, SQ, DM), jnp.float32),
        in_specs=[
            pl.BlockSpec(memory_space=pl.ANY),
            pl.BlockSpec(memory_space=pltpu.VMEM),
            pl.BlockSpec(memory_space=pl.ANY),
            pl.BlockSpec(memory_space=pl.ANY),
            pl.BlockSpec(memory_space=pltpu.VMEM),
        ],
        out_specs=pl.BlockSpec(memory_space=pltpu.VMEM),
        scratch_shapes=[
            pltpu.VMEM((SQ, DM), jnp.bfloat16),
            pltpu.VMEM((KBUF, DM), jnp.bfloat16),
            pltpu.VMEM((KBUF, DM), jnp.bfloat16),
            pltpu.VMEM((GLOB, DM), jnp.bfloat16),
            pltpu.VMEM((SQ, DM), jnp.bfloat16),
            pltpu.VMEM((CHUNK, DM), jnp.float32),
            pltpu.VMEM((CHUNK, HQ, DH), jnp.float32),
            pltpu.VMEM((HQ * GLOB, DH), jnp.float32),
            pltpu.VMEM((HQ * GLOB, DH), jnp.float32),
            pltpu.VMEM((HQ * GLOB, DH), jnp.float32),
            pltpu.VMEM((3, HQ * GLOB, DH), jnp.float32),
            pltpu.VMEM((3, HQ * GLOB, DH), jnp.float32),
            pltpu.VMEM((3, HQ * GLOB, DH), jnp.float32),
            pltpu.SemaphoreType.DMA,
            pltpu.SemaphoreType.DMA((4,)),
            pltpu.SemaphoreType.DMA((4,)),
            pltpu.SemaphoreType.DMA((9,)),
            pltpu.SemaphoreType.DMA((3,)),
            pltpu.SemaphoreType.DMA((3,)),
            pltpu.SemaphoreType.DMA((3, 3)),
        ],
        compiler_params=pltpu.CompilerParams(
            collective_id=0, vmem_limit_bytes=46 * 1024 * 1024),
    )(x, Wq, K_ext, V_ext, Wo)


# device time: 97856 ns/iter; 1.7734x vs baseline; 1.1648x over previous
import jax
import jax.numpy as jnp
from jax import lax
from jax.experimental import pallas as pl
from jax.experimental.pallas import tpu as pltpu

N_DEV = 4
SQ = 2048
HQ = 8
DH = 128
DM = HQ * DH
HALO = 128
GLOB = 32
GSLOT = 128
OWN = GSLOT + HALO
KBUF = GSLOT + HALO + SQ + HALO
QBLK = 256
CHUNK = 512
SCALE = 0.08838834764831843
NEG = -1e9


def kernel(x, Wq, K_ext, V_ext, Wo):
    def body(x_ref, wq_ref, k_ref, v_ref, wo_ref, out_ref,
             qbuf, kbuf, vbuf, qg, ctx, xstage, kstage, vstage,
             po, pm, plb, rxo, rxm, rxl,
             lsem, ksem, vsem, halo_send, halo_recv, gsend, grecv,
             psend, precv):
        my = lax.axis_index("i")
        left = lax.rem(my + N_DEV - 1, N_DEV)
        right = lax.rem(my + 1, N_DEV)

        bsem = pltpu.get_barrier_semaphore()
        for nbr in (left, right):
            pl.semaphore_signal(bsem, inc=1, device_id=(nbr,),
                                device_id_type=pl.DeviceIdType.MESH)
        on_diag = jnp.logical_or(my == 0, my == 2)

        @pl.when(on_diag)
        def _():
            other = jnp.where(my == 0, 2, 0)
            pl.semaphore_signal(bsem, inc=1, device_id=(other,),
                                device_id_type=pl.DeviceIdType.MESH)
            pl.semaphore_wait(bsem, 3)

        @pl.when(jnp.logical_not(on_diag))
        def _():
            pl.semaphore_wait(bsem, 2)

        wqb = wq_ref[...].astype(jnp.bfloat16)
        NC = SQ // CHUNK
        xc = [pltpu.make_async_copy(
            x_ref.at[0, pl.ds(c * CHUNK, CHUNK)], xstage.at[c % 2],
            lsem.at[c % 2]) for c in range(NC)]
        kc = [pltpu.make_async_copy(
            k_ref.at[0, pl.ds(c * CHUNK, CHUNK)], kstage.at[c % 2],
            ksem.at[c % 2]) for c in range(NC)]
        vc = [pltpu.make_async_copy(
            v_ref.at[0, pl.ds(c * CHUNK, CHUNK)], vstage.at[c % 2],
            vsem.at[c % 2]) for c in range(NC)]
        for d in (xc[0], xc[1], kc[0], kc[1], vc[0], vc[1]):
            d.start()
        for c in range(NC):
            xc[c].wait()
            qbuf[pl.ds(c * CHUNK, CHUNK), :] = (lax.dot(
                xstage[c % 2].astype(jnp.bfloat16), wqb,
                preferred_element_type=jnp.float32)
                * SCALE).astype(jnp.bfloat16)
            if c + 2 < NC:
                xc[c + 2].start()
            kc[c].wait()
            kbuf[pl.ds(OWN + c * CHUNK, CHUNK), :] = \
                kstage[c % 2].reshape(CHUNK, DM).astype(jnp.bfloat16)
            if c + 2 < NC:
                kc[c + 2].start()
            vc[c].wait()
            vbuf[pl.ds(OWN + c * CHUNK, CHUNK), :] = \
                vstage[c % 2].reshape(CHUNK, DM).astype(jnp.bfloat16)
            if c + 2 < NC:
                vc[c + 2].start()
        kbuf[GLOB:GSLOT, :] = jnp.zeros((GSLOT - GLOB, DM), jnp.bfloat16)
        vbuf[GLOB:GSLOT, :] = jnp.zeros((GSLOT - GLOB, DM), jnp.bfloat16)

        @pl.when(my == 0)
        def _():
            kbuf[0:GLOB, :] = kbuf[OWN:OWN + GLOB, :]
            vbuf[0:GLOB, :] = vbuf[OWN:OWN + GLOB, :]
            qg[...] = qbuf[0:GLOB, :]

        halo_rdmas = []
        for buf, s0 in ((kbuf, 0), (vbuf, 2)):
            halo_rdmas.append(pltpu.make_async_remote_copy(
                src_ref=buf.at[pl.ds(OWN, HALO)],
                dst_ref=buf.at[pl.ds(OWN + SQ, HALO)],
                send_sem=halo_send.at[s0], recv_sem=halo_recv.at[s0],
                device_id=(left,), device_id_type=pl.DeviceIdType.MESH))
            halo_rdmas.append(pltpu.make_async_remote_copy(
                src_ref=buf.at[pl.ds(OWN + SQ - HALO, HALO)],
                dst_ref=buf.at[pl.ds(GSLOT, HALO)],
                send_sem=halo_send.at[s0 + 1], recv_sem=halo_recv.at[s0 + 1],
                device_id=(right,), device_id_type=pl.DeviceIdType.MESH))
        for r in halo_rdmas:
            r.start()

        @pl.when(my == 0)
        def _():
            sends = []
            i = 0
            for dst in (1, 2, 3):
                for src_r, dst_r, j in (
                        (kbuf.at[pl.ds(OWN, GLOB)], kbuf.at[pl.ds(0, GLOB)], 0),
                        (vbuf.at[pl.ds(OWN, GLOB)], vbuf.at[pl.ds(0, GLOB)], 1),
                        (qbuf.at[pl.ds(0, GLOB)], qg, 2)):
                    d = pltpu.make_async_remote_copy(
                        src_ref=src_r, dst_ref=dst_r,
                        send_sem=gsend.at[i], recv_sem=grecv.at[j],
                        device_id=(dst,), device_id_type=pl.DeviceIdType.MESH)
                    d.start()
                    sends.append(d)
                    i += 1
            for d in sends:
                d.wait_send()

        def recv_only(dst_r, rsem):
            return pltpu.make_async_remote_copy(
                src_ref=dst_r, dst_ref=dst_r, send_sem=gsend.at[0],
                recv_sem=rsem, device_id=(0,),
                device_id_type=pl.DeviceIdType.MESH)

        @pl.when(my != 0)
        def _():
            recv_only(qg, grecv.at[2]).wait_recv()

        qgb = qg[...]
        for h in range(HQ):
            qh = qgb[:, h * DH:(h + 1) * DH]
            s = lax.dot_general(qh, kbuf[OWN:OWN + SQ, h * DH:(h + 1) * DH],
                                (((1,), (1,)), ((), ())),
                                preferred_element_type=jnp.float32)
            m = jnp.max(s, axis=1, keepdims=True)
            w = jnp.exp(s - m)
            l = jnp.sum(w, axis=1, keepdims=True)
            o = lax.dot_general(w.astype(jnp.bfloat16),
                                vbuf[OWN:OWN + SQ, h * DH:(h + 1) * DH],
                                (((1,), (0,)), ((), ())),
                                preferred_element_type=jnp.float32)
            po[h * GLOB:(h + 1) * GLOB, :] = o
            pm[h * GLOB:(h + 1) * GLOB, :] = jnp.broadcast_to(m, (GLOB, DH))
            plb[h * GLOB:(h + 1) * GLOB, :] = jnp.broadcast_to(l, (GLOB, DH))

        for src in (1, 2, 3):
            @pl.when(my == src)
            def _(src=src):
                ds = []
                for j, (sbuf, rbuf) in enumerate(
                        ((po, rxo), (pm, rxm), (plb, rxl))):
                    d = pltpu.make_async_remote_copy(
                        src_ref=sbuf, dst_ref=rbuf.at[src - 1],
                        send_sem=psend.at[j], recv_sem=precv.at[src - 1, j],
                        device_id=(0,), device_id_type=pl.DeviceIdType.MESH)
                    d.start()
                    ds.append(d)
                for d in ds:
                    d.wait_send()

        for r in halo_rdmas:
            r.wait()

        @pl.when(my != 0)
        def _():
            recv_only(kbuf.at[pl.ds(0, GLOB)], grecv.at[0]).wait_recv()
            recv_only(vbuf.at[pl.ds(0, GLOB)], grecv.at[1]).wait_recv()

        WWIN = QBLK + 2 * HALO
        cg = lax.broadcasted_iota(jnp.int32, (1, GSLOT), 1)
        bias_g = jnp.where(cg < GLOB, 0.0, NEG).astype(jnp.float32)

        def qb_body(qb, carry):
            q0 = qb * QBLK
            cw = lax.broadcasted_iota(jnp.int32, (1, WWIN), 1)
            ciw = GSLOT + q0 + cw
            kiw = my * SQ + ciw - OWN
            qi = my * SQ + q0 + lax.broadcasted_iota(jnp.int32, (QBLK, 1), 0)
            band = (kiw >= qi - HALO) & (kiw <= qi + HALO) & (kiw >= GLOB)
            band = band & jnp.logical_not(
                jnp.logical_and(my == N_DEV - 1, ciw >= OWN + SQ))
            bias = jnp.where(band, 0.0, NEG).astype(jnp.float32)

            def h_body(h, hc):
                c0 = h * DH
                qh = qbuf[pl.ds(q0, QBLK), pl.ds(c0, DH)]
                sw = lax.dot_general(qh, kbuf[pl.ds(GSLOT + q0, WWIN),
                                              pl.ds(c0, DH)],
                                     (((1,), (1,)), ((), ())),
                                     preferred_element_type=jnp.float32)
                sw = sw + bias
                sg = lax.dot_general(qh, kbuf[0:GSLOT, pl.ds(c0, DH)],
                                     (((1,), (1,)), ((), ())),
                                     preferred_element_type=jnp.float32)
                sg = sg + bias_g
                m = jnp.maximum(jnp.max(sw, axis=1, keepdims=True),
                                jnp.max(sg, axis=1, keepdims=True))
                ww = jnp.exp(sw - m)
                wg = jnp.exp(sg - m)
                l = (jnp.sum(ww, axis=1, keepdims=True)
                     + jnp.sum(wg, axis=1, keepdims=True))
                o = lax.dot_general(ww.astype(jnp.bfloat16),
                                    vbuf[pl.ds(GSLOT + q0, WWIN),
                                         pl.ds(c0, DH)],
                                    (((1,), (0,)), ((), ())),
                                    preferred_element_type=jnp.float32)
                o = o + lax.dot_general(wg.astype(jnp.bfloat16),
                                        vbuf[0:GSLOT, pl.ds(c0, DH)],
                                        (((1,), (0,)), ((), ())),
                                        preferred_element_type=jnp.float32)
                ctx[pl.ds(q0, QBLK), pl.ds(c0, DH)] = \
                    (o / l).astype(jnp.bfloat16)
                return hc

            lax.fori_loop(0, HQ, h_body, 0)
            return carry

        lax.fori_loop(0, SQ // QBLK, qb_body, 0)

        @pl.when(my == 0)
        def _():
            for s in range(3):
                for j, rbuf in enumerate((rxo, rxm, rxl)):
                    pltpu.make_async_remote_copy(
                        src_ref=rbuf.at[s], dst_ref=rbuf.at[s],
                        send_sem=psend.at[0], recv_sem=precv.at[s, j],
                        device_id=(0,),
                        device_id_type=pl.DeviceIdType.MESH).wait_recv()
            m0 = pm[...]
            m1, m2, m3 = rxm[0], rxm[1], rxm[2]
            mx = jnp.maximum(jnp.maximum(m0, m1), jnp.maximum(m2, m3))
            e0 = jnp.exp(m0 - mx)
            e1 = jnp.exp(m1 - mx)
            e2 = jnp.exp(m2 - mx)
            e3 = jnp.exp(m3 - mx)
            osum = po[...] * e0 + rxo[0] * e1 + rxo[1] * e2 + rxo[2] * e3
            lsum = plb[...] * e0 + rxl[0] * e1 + rxl[1] * e2 + rxl[2] * e3
            cg = (osum / lsum).astype(jnp.bfloat16)
            for h in range(HQ):
                ctx[0:GLOB, h * DH:(h + 1) * DH] = cg[h * GLOB:(h + 1) * GLOB, :]

        wob = wo_ref[...].astype(jnp.bfloat16)
        for c in range(SQ // CHUNK):
            out_ref[0, pl.ds(c * CHUNK, CHUNK), :] = lax.dot(
                ctx[pl.ds(c * CHUNK, CHUNK), :], wob,
                preferred_element_type=jnp.float32)

    return pl.pallas_call(
        body,
        out_shape=jax.ShapeDtypeStruct((1, SQ, DM), jnp.float32),
        in_specs=[
            pl.BlockSpec(memory_space=pl.ANY),
            pl.BlockSpec(memory_space=pltpu.VMEM),
            pl.BlockSpec(memory_space=pl.ANY),
            pl.BlockSpec(memory_space=pl.ANY),
            pl.BlockSpec(memory_space=pltpu.VMEM),
        ],
        out_specs=pl.BlockSpec(memory_space=pltpu.VMEM),
        scratch_shapes=[
            pltpu.VMEM((SQ, DM), jnp.bfloat16),
            pltpu.VMEM((KBUF, DM), jnp.bfloat16),
            pltpu.VMEM((KBUF, DM), jnp.bfloat16),
            pltpu.VMEM((GLOB, DM), jnp.bfloat16),
            pltpu.VMEM((SQ, DM), jnp.bfloat16),
            pltpu.VMEM((2, CHUNK, DM), jnp.float32),
            pltpu.VMEM((2, CHUNK, HQ, DH), jnp.float32),
            pltpu.VMEM((2, CHUNK, HQ, DH), jnp.float32),
            pltpu.VMEM((HQ * GLOB, DH), jnp.float32),
            pltpu.VMEM((HQ * GLOB, DH), jnp.float32),
            pltpu.VMEM((HQ * GLOB, DH), jnp.float32),
            pltpu.VMEM((3, HQ * GLOB, DH), jnp.float32),
            pltpu.VMEM((3, HQ * GLOB, DH), jnp.float32),
            pltpu.VMEM((3, HQ * GLOB, DH), jnp.float32),
            pltpu.SemaphoreType.DMA((2,)),
            pltpu.SemaphoreType.DMA((2,)),
            pltpu.SemaphoreType.DMA((2,)),
            pltpu.SemaphoreType.DMA((4,)),
            pltpu.SemaphoreType.DMA((4,)),
            pltpu.SemaphoreType.DMA((9,)),
            pltpu.SemaphoreType.DMA((3,)),
            pltpu.SemaphoreType.DMA((3,)),
            pltpu.SemaphoreType.DMA((3, 3)),
        ],
        compiler_params=pltpu.CompilerParams(
            collective_id=0, vmem_limit_bytes=47 * 1024 * 1024),
    )(x, Wq, K_ext, V_ext, Wo)


# device time: 85153 ns/iter; 2.0380x vs baseline; 1.1492x over previous
import jax
import jax.numpy as jnp
from jax import lax
from jax.experimental import pallas as pl
from jax.experimental.pallas import tpu as pltpu

N_DEV = 4
SQ = 2048
HQ = 8
DH = 128
DM = HQ * DH
HALO = 128
GLOB = 32
GSLOT = 128
OWN = GSLOT + HALO
KBUF = GSLOT + HALO + SQ + HALO
QBLK = 256
CHUNK = 512
SCALE = 0.08838834764831843
NEG = -1e9


def kernel(x, Wq, K_ext, V_ext, Wo):
    def body(x_ref, wq_ref, k_ref, v_ref, wo_ref, out_ref,
             qbuf, kbuf, vbuf, qg, ctx, xstage, kstage, vstage,
             po, plb, rxo, rxl,
             lsem, ksem, vsem, halo_send, halo_recv, gsend, grecv,
             psend, precv):
        my = lax.axis_index("i")
        left = lax.rem(my + N_DEV - 1, N_DEV)
        right = lax.rem(my + 1, N_DEV)

        bsem = pltpu.get_barrier_semaphore()
        for nbr in (left, right):
            pl.semaphore_signal(bsem, inc=1, device_id=(nbr,),
                                device_id_type=pl.DeviceIdType.MESH)
        on_diag = jnp.logical_or(my == 0, my == 2)

        @pl.when(on_diag)
        def _():
            other = jnp.where(my == 0, 2, 0)
            pl.semaphore_signal(bsem, inc=1, device_id=(other,),
                                device_id_type=pl.DeviceIdType.MESH)
            pl.semaphore_wait(bsem, 3)

        @pl.when(jnp.logical_not(on_diag))
        def _():
            pl.semaphore_wait(bsem, 2)

        wqb = wq_ref[...].astype(jnp.bfloat16)
        NC = SQ // CHUNK
        xc = [pltpu.make_async_copy(
            x_ref.at[0, pl.ds(c * CHUNK, CHUNK)], xstage.at[c % 2],
            lsem.at[c % 2]) for c in range(NC)]
        kc = [pltpu.make_async_copy(
            k_ref.at[0, pl.ds(c * CHUNK, CHUNK)], kstage.at[c % 2],
            ksem.at[c % 2]) for c in range(NC)]
        vc = [pltpu.make_async_copy(
            v_ref.at[0, pl.ds(c * CHUNK, CHUNK)], vstage.at[c % 2],
            vsem.at[c % 2]) for c in range(NC)]
        for d in (xc[0], xc[1], kc[0], kc[1], vc[0], vc[1]):
            d.start()
        for c in range(NC):
            xc[c].wait()
            qbuf[pl.ds(c * CHUNK, CHUNK), :] = (lax.dot(
                xstage[c % 2].astype(jnp.bfloat16), wqb,
                preferred_element_type=jnp.float32)
                * SCALE).astype(jnp.bfloat16)
            if c + 2 < NC:
                xc[c + 2].start()
            kc[c].wait()
            kbuf[pl.ds(OWN + c * CHUNK, CHUNK), :] = \
                kstage[c % 2].reshape(CHUNK, DM).astype(jnp.bfloat16)
            if c + 2 < NC:
                kc[c + 2].start()
            vc[c].wait()
            vbuf[pl.ds(OWN + c * CHUNK, CHUNK), :] = \
                vstage[c % 2].reshape(CHUNK, DM).astype(jnp.bfloat16)
            if c + 2 < NC:
                vc[c + 2].start()
        @pl.when(my == 0)
        def _():
            kbuf[0:GLOB, :] = kbuf[OWN:OWN + GLOB, :]
            vbuf[0:GLOB, :] = vbuf[OWN:OWN + GLOB, :]
            qg[...] = qbuf[0:GLOB, :]

        halo_rdmas = []
        for buf, s0 in ((kbuf, 0), (vbuf, 2)):
            halo_rdmas.append(pltpu.make_async_remote_copy(
                src_ref=buf.at[pl.ds(OWN, HALO)],
                dst_ref=buf.at[pl.ds(OWN + SQ, HALO)],
                send_sem=halo_send.at[s0], recv_sem=halo_recv.at[s0],
                device_id=(left,), device_id_type=pl.DeviceIdType.MESH))
            halo_rdmas.append(pltpu.make_async_remote_copy(
                src_ref=buf.at[pl.ds(OWN + SQ - HALO, HALO)],
                dst_ref=buf.at[pl.ds(GSLOT, HALO)],
                send_sem=halo_send.at[s0 + 1], recv_sem=halo_recv.at[s0 + 1],
                device_id=(right,), device_id_type=pl.DeviceIdType.MESH))
        for r in halo_rdmas:
            r.start()

        @pl.when(my == 0)
        def _():
            sends = []
            i = 0
            for dst in (1, 2, 3):
                for src_r, dst_r, j in (
                        (kbuf.at[pl.ds(OWN, GLOB)], kbuf.at[pl.ds(0, GLOB)], 0),
                        (vbuf.at[pl.ds(OWN, GLOB)], vbuf.at[pl.ds(0, GLOB)], 1),
                        (qbuf.at[pl.ds(0, GLOB)], qg, 2)):
                    d = pltpu.make_async_remote_copy(
                        src_ref=src_r, dst_ref=dst_r,
                        send_sem=gsend.at[i], recv_sem=grecv.at[j],
                        device_id=(dst,), device_id_type=pl.DeviceIdType.MESH)
                    d.start()
                    sends.append(d)
                    i += 1
            for d in sends:
                d.wait_send()

        def recv_only(dst_r, rsem):
            return pltpu.make_async_remote_copy(
                src_ref=dst_r, dst_ref=dst_r, send_sem=gsend.at[0],
                recv_sem=rsem, device_id=(0,),
                device_id_type=pl.DeviceIdType.MESH)

        @pl.when(my != 0)
        def _():
            recv_only(qg, grecv.at[2]).wait_recv()

        qgb = qg[...]
        for h in range(HQ):
            qh = qgb[:, h * DH:(h + 1) * DH]
            s = lax.dot_general(qh, kbuf[OWN:OWN + SQ, h * DH:(h + 1) * DH],
                                (((1,), (1,)), ((), ())),
                                preferred_element_type=jnp.float32)
            w = jnp.exp(s)
            l = jnp.sum(w, axis=1, keepdims=True)
            o = lax.dot_general(w.astype(jnp.bfloat16),
                                vbuf[OWN:OWN + SQ, h * DH:(h + 1) * DH],
                                (((1,), (0,)), ((), ())),
                                preferred_element_type=jnp.float32)
            po[h * GLOB:(h + 1) * GLOB, :] = o
            plb[h * GLOB:(h + 1) * GLOB, :] = jnp.broadcast_to(l, (GLOB, DH))

        for src in (1, 2, 3):
            @pl.when(my == src)
            def _(src=src):
                ds = []
                for j, (sbuf, rbuf) in enumerate(((po, rxo), (plb, rxl))):
                    d = pltpu.make_async_remote_copy(
                        src_ref=sbuf, dst_ref=rbuf.at[src - 1],
                        send_sem=psend.at[j], recv_sem=precv.at[src - 1, j],
                        device_id=(0,), device_id_type=pl.DeviceIdType.MESH)
                    d.start()
                    ds.append(d)
                for d in ds:
                    d.wait_send()

        for r in halo_rdmas:
            r.wait()

        @pl.when(my != 0)
        def _():
            recv_only(kbuf.at[pl.ds(0, GLOB)], grecv.at[0]).wait_recv()
            recv_only(vbuf.at[pl.ds(0, GLOB)], grecv.at[1]).wait_recv()

        WWIN = QBLK + 2 * HALO

        def qb_body(qb, carry):
            q0 = qb * QBLK
            cw = lax.broadcasted_iota(jnp.int32, (1, WWIN), 1)
            ciw = GSLOT + q0 + cw
            kiw = my * SQ + ciw - OWN
            qi = my * SQ + q0 + lax.broadcasted_iota(jnp.int32, (QBLK, 1), 0)
            band = (kiw >= qi - HALO) & (kiw <= qi + HALO) & (kiw >= GLOB)
            band = band & jnp.logical_not(
                jnp.logical_and(my == N_DEV - 1, ciw >= OWN + SQ))
            bias = jnp.where(band, 0.0, NEG).astype(jnp.float32)

            def h_body(h, hc):
                c0 = h * DH
                qh = qbuf[pl.ds(q0, QBLK), pl.ds(c0, DH)]
                sw = lax.dot_general(qh, kbuf[pl.ds(GSLOT + q0, WWIN),
                                              pl.ds(c0, DH)],
                                     (((1,), (1,)), ((), ())),
                                     preferred_element_type=jnp.float32)
                ww = jnp.exp(sw + bias)
                sg = lax.dot_general(qh, kbuf[0:GLOB, pl.ds(c0, DH)],
                                     (((1,), (1,)), ((), ())),
                                     preferred_element_type=jnp.float32)
                wg = jnp.exp(sg)
                l = (jnp.sum(ww, axis=1, keepdims=True)
                     + jnp.sum(wg, axis=1, keepdims=True))
                o = lax.dot_general(ww.astype(jnp.bfloat16),
                                    vbuf[pl.ds(GSLOT + q0, WWIN),
                                         pl.ds(c0, DH)],
                                    (((1,), (0,)), ((), ())),
                                    preferred_element_type=jnp.float32)
                o = o + lax.dot_general(wg.astype(jnp.bfloat16),
                                        vbuf[0:GLOB, pl.ds(c0, DH)],
                                        (((1,), (0,)), ((), ())),
                                        preferred_element_type=jnp.float32)
                ctx[pl.ds(q0, QBLK), pl.ds(c0, DH)] = \
                    (o / l).astype(jnp.bfloat16)
                return hc

            lax.fori_loop(0, HQ, h_body, 0)
            return carry

        lax.fori_loop(0, SQ // QBLK, qb_body, 0)

        @pl.when(my == 0)
        def _():
            for s in range(3):
                for j, rbuf in enumerate((rxo, rxl)):
                    pltpu.make_async_remote_copy(
                        src_ref=rbuf.at[s], dst_ref=rbuf.at[s],
                        send_sem=psend.at[0], recv_sem=precv.at[s, j],
                        device_id=(0,),
                        device_id_type=pl.DeviceIdType.MESH).wait_recv()
            osum = po[...] + rxo[0] + rxo[1] + rxo[2]
            lsum = plb[...] + rxl[0] + rxl[1] + rxl[2]
            cg = (osum / lsum).astype(jnp.bfloat16)
            for h in range(HQ):
                ctx[0:GLOB, h * DH:(h + 1) * DH] = cg[h * GLOB:(h + 1) * GLOB, :]

        wob = wo_ref[...].astype(jnp.bfloat16)
        for c in range(SQ // CHUNK):
            out_ref[0, pl.ds(c * CHUNK, CHUNK), :] = lax.dot(
                ctx[pl.ds(c * CHUNK, CHUNK), :], wob,
                preferred_element_type=jnp.float32)

    return pl.pallas_call(
        body,
        out_shape=jax.ShapeDtypeStruct((1, SQ, DM), jnp.float32),
        in_specs=[
            pl.BlockSpec(memory_space=pl.ANY),
            pl.BlockSpec(memory_space=pltpu.VMEM),
            pl.BlockSpec(memory_space=pl.ANY),
            pl.BlockSpec(memory_space=pl.ANY),
            pl.BlockSpec(memory_space=pltpu.VMEM),
        ],
        out_specs=pl.BlockSpec(memory_space=pltpu.VMEM),
        scratch_shapes=[
            pltpu.VMEM((SQ, DM), jnp.bfloat16),
            pltpu.VMEM((KBUF, DM), jnp.bfloat16),
            pltpu.VMEM((KBUF, DM), jnp.bfloat16),
            pltpu.VMEM((GLOB, DM), jnp.bfloat16),
            pltpu.VMEM((SQ, DM), jnp.bfloat16),
            pltpu.VMEM((2, CHUNK, DM), jnp.float32),
            pltpu.VMEM((2, CHUNK, HQ, DH), jnp.float32),
            pltpu.VMEM((2, CHUNK, HQ, DH), jnp.float32),
            pltpu.VMEM((HQ * GLOB, DH), jnp.float32),
            pltpu.VMEM((HQ * GLOB, DH), jnp.float32),
            pltpu.VMEM((3, HQ * GLOB, DH), jnp.float32),
            pltpu.VMEM((3, HQ * GLOB, DH), jnp.float32),
            pltpu.SemaphoreType.DMA((2,)),
            pltpu.SemaphoreType.DMA((2,)),
            pltpu.SemaphoreType.DMA((2,)),
            pltpu.SemaphoreType.DMA((4,)),
            pltpu.SemaphoreType.DMA((4,)),
            pltpu.SemaphoreType.DMA((9,)),
            pltpu.SemaphoreType.DMA((3,)),
            pltpu.SemaphoreType.DMA((2,)),
            pltpu.SemaphoreType.DMA((3, 2)),
        ],
        compiler_params=pltpu.CompilerParams(
            collective_id=0, vmem_limit_bytes=47 * 1024 * 1024),
    )(x, Wq, K_ext, V_ext, Wo)


# device time: 67448 ns/iter; 2.5730x vs baseline; 1.2625x over previous
import jax
import jax.numpy as jnp
from jax import lax
from jax.experimental import pallas as pl
from jax.experimental.pallas import tpu as pltpu

N_DEV = 4
SQ = 2048
HQ = 8
DH = 128
DM = HQ * DH
HALO = 128
GLOB = 32
GSLOT = 128
OWN = GSLOT + HALO
KBUF = GSLOT + HALO + SQ + HALO
QBLK = 256
CHUNK = 512
SCALE = 0.08838834764831843
NEG = -1e9


def kernel(x, Wq, K_ext, V_ext, Wo):
    def body(x_ref, wq_ref, k_ref, v_ref, wo_ref, out_ref,
             qbuf, kbuf, vbuf, qg, ctx, xstage, kstage, vstage,
             po, plb, rxo, rxl,
             lsem, ksem, vsem, halo_send, halo_recv, gsend, grecv,
             psend, precv):
        my = lax.axis_index("i")
        left = lax.rem(my + N_DEV - 1, N_DEV)
        right = lax.rem(my + 1, N_DEV)

        bsem = pltpu.get_barrier_semaphore()
        for nbr in (left, right):
            pl.semaphore_signal(bsem, inc=1, device_id=(nbr,),
                                device_id_type=pl.DeviceIdType.MESH)
        on_diag = jnp.logical_or(my == 0, my == 2)

        @pl.when(on_diag)
        def _():
            other = jnp.where(my == 0, 2, 0)
            pl.semaphore_signal(bsem, inc=1, device_id=(other,),
                                device_id_type=pl.DeviceIdType.MESH)
            pl.semaphore_wait(bsem, 3)

        @pl.when(jnp.logical_not(on_diag))
        def _():
            pl.semaphore_wait(bsem, 2)

        wqb = wq_ref[...].astype(jnp.bfloat16)
        NC = SQ // CHUNK
        xc = [pltpu.make_async_copy(
            x_ref.at[0, pl.ds(c * CHUNK, CHUNK)], xstage.at[c % 2],
            lsem.at[c % 2]) for c in range(NC)]
        kc = [pltpu.make_async_copy(
            k_ref.at[0, pl.ds(c * CHUNK, CHUNK)], kstage.at[c % 2],
            ksem.at[c % 2]) for c in range(NC)]
        vc = [pltpu.make_async_copy(
            v_ref.at[0, pl.ds(c * CHUNK, CHUNK)], vstage.at[c % 2],
            vsem.at[c % 2]) for c in range(NC)]
        for d in (xc[0], xc[1], kc[0], kc[1], vc[0], vc[1]):
            d.start()
        for c in range(NC):
            xc[c].wait()
            qbuf[pl.ds(c * CHUNK, CHUNK), :] = (lax.dot(
                xstage[c % 2].astype(jnp.bfloat16), wqb,
                preferred_element_type=jnp.float32)
                * SCALE).astype(jnp.bfloat16)
            if c + 2 < NC:
                xc[c + 2].start()
            kc[c].wait()
            kbuf[pl.ds(OWN + c * CHUNK, CHUNK), :] = \
                kstage[c % 2].reshape(CHUNK, DM).astype(jnp.bfloat16)
            if c + 2 < NC:
                kc[c + 2].start()
            vc[c].wait()
            vbuf[pl.ds(OWN + c * CHUNK, CHUNK), :] = \
                vstage[c % 2].reshape(CHUNK, DM).astype(jnp.bfloat16)
            if c + 2 < NC:
                vc[c + 2].start()
        @pl.when(my == 0)
        def _():
            kbuf[0:GLOB, :] = kbuf[OWN:OWN + GLOB, :]
            vbuf[0:GLOB, :] = vbuf[OWN:OWN + GLOB, :]
            qg[...] = qbuf[0:GLOB, :]

        halo_rdmas = []
        for buf, s0 in ((kbuf, 0), (vbuf, 2)):
            halo_rdmas.append(pltpu.make_async_remote_copy(
                src_ref=buf.at[pl.ds(OWN, HALO)],
                dst_ref=buf.at[pl.ds(OWN + SQ, HALO)],
                send_sem=halo_send.at[s0], recv_sem=halo_recv.at[s0],
                device_id=(left,), device_id_type=pl.DeviceIdType.MESH))
            halo_rdmas.append(pltpu.make_async_remote_copy(
                src_ref=buf.at[pl.ds(OWN + SQ - HALO, HALO)],
                dst_ref=buf.at[pl.ds(GSLOT, HALO)],
                send_sem=halo_send.at[s0 + 1], recv_sem=halo_recv.at[s0 + 1],
                device_id=(right,), device_id_type=pl.DeviceIdType.MESH))
        for r in halo_rdmas:
            r.start()

        @pl.when(my == 0)
        def _():
            sends = []
            i = 0
            for dst in (1, 2, 3):
                for src_r, dst_r, j in (
                        (kbuf.at[pl.ds(OWN, GLOB)], kbuf.at[pl.ds(0, GLOB)], 0),
                        (vbuf.at[pl.ds(OWN, GLOB)], vbuf.at[pl.ds(0, GLOB)], 1),
                        (qbuf.at[pl.ds(0, GLOB)], qg, 2)):
                    d = pltpu.make_async_remote_copy(
                        src_ref=src_r, dst_ref=dst_r,
                        send_sem=gsend.at[i], recv_sem=grecv.at[j],
                        device_id=(dst,), device_id_type=pl.DeviceIdType.MESH)
                    d.start()
                    sends.append(d)
                    i += 1
            for d in sends:
                d.wait_send()

        def recv_only(dst_r, rsem):
            return pltpu.make_async_remote_copy(
                src_ref=dst_r, dst_ref=dst_r, send_sem=gsend.at[0],
                recv_sem=rsem, device_id=(0,),
                device_id_type=pl.DeviceIdType.MESH)

        @pl.when(my != 0)
        def _():
            recv_only(qg, grecv.at[2]).wait_recv()

        qgb = qg[...]
        for h in range(HQ):
            qh = qgb[:, h * DH:(h + 1) * DH]
            s = lax.dot_general(qh, kbuf[OWN:OWN + SQ, h * DH:(h + 1) * DH],
                                (((1,), (1,)), ((), ())),
                                preferred_element_type=jnp.float32)
            w = jnp.exp(s)
            l = jnp.sum(w, axis=1, keepdims=True)
            o = lax.dot_general(w.astype(jnp.bfloat16),
                                vbuf[OWN:OWN + SQ, h * DH:(h + 1) * DH],
                                (((1,), (0,)), ((), ())),
                                preferred_element_type=jnp.float32)
            po[h * GLOB:(h + 1) * GLOB, :] = o
            plb[h * GLOB:(h + 1) * GLOB, :] = jnp.broadcast_to(l, (GLOB, DH))

        for src in (1, 2, 3):
            @pl.when(my == src)
            def _(src=src):
                ds = []
                for j, (sbuf, rbuf) in enumerate(((po, rxo), (plb, rxl))):
                    d = pltpu.make_async_remote_copy(
                        src_ref=sbuf, dst_ref=rbuf.at[src - 1],
                        send_sem=psend.at[j], recv_sem=precv.at[src - 1, j],
                        device_id=(0,), device_id_type=pl.DeviceIdType.MESH)
                    d.start()
                    ds.append(d)
                for d in ds:
                    d.wait_send()

        for r in halo_rdmas:
            r.wait()

        @pl.when(my != 0)
        def _():
            recv_only(kbuf.at[pl.ds(0, GLOB)], grecv.at[0]).wait_recv()
            recv_only(vbuf.at[pl.ds(0, GLOB)], grecv.at[1]).wait_recv()

        WWIN = QBLK + 2 * HALO

        def qb_body(qb, carry):
            q0 = qb * QBLK
            cw = lax.broadcasted_iota(jnp.int32, (1, WWIN), 1)
            ciw = GSLOT + q0 + cw
            kiw = my * SQ + ciw - OWN
            qi = my * SQ + q0 + lax.broadcasted_iota(jnp.int32, (QBLK, 1), 0)
            band = (kiw >= qi - HALO) & (kiw <= qi + HALO) & (kiw >= GLOB)
            band = band & jnp.logical_not(
                jnp.logical_and(my == N_DEV - 1, ciw >= OWN + SQ))
            bias = jnp.where(band, 0.0, NEG).astype(jnp.float32)

            for h in range(HQ):
                c0 = h * DH
                qh = qbuf[pl.ds(q0, QBLK), pl.ds(c0, DH)]
                sw = lax.dot_general(qh, kbuf[pl.ds(GSLOT + q0, WWIN),
                                              pl.ds(c0, DH)],
                                     (((1,), (1,)), ((), ())),
                                     preferred_element_type=jnp.float32)
                ww = jnp.exp(sw + bias)
                sg = lax.dot_general(qh, kbuf[0:GLOB, pl.ds(c0, DH)],
                                     (((1,), (1,)), ((), ())),
                                     preferred_element_type=jnp.float32)
                wg = jnp.exp(sg)
                l = (jnp.sum(ww, axis=1, keepdims=True)
                     + jnp.sum(wg, axis=1, keepdims=True))
                o = lax.dot_general(ww.astype(jnp.bfloat16),
                                    vbuf[pl.ds(GSLOT + q0, WWIN),
                                         pl.ds(c0, DH)],
                                    (((1,), (0,)), ((), ())),
                                    preferred_element_type=jnp.float32)
                o = o + lax.dot_general(wg.astype(jnp.bfloat16),
                                        vbuf[0:GLOB, pl.ds(c0, DH)],
                                        (((1,), (0,)), ((), ())),
                                        preferred_element_type=jnp.float32)
                ctx[pl.ds(q0, QBLK), pl.ds(c0, DH)] = \
                    (o / l).astype(jnp.bfloat16)
            return carry

        lax.fori_loop(0, SQ // QBLK, qb_body, 0)

        @pl.when(my == 0)
        def _():
            for s in range(3):
                for j, rbuf in enumerate((rxo, rxl)):
                    pltpu.make_async_remote_copy(
                        src_ref=rbuf.at[s], dst_ref=rbuf.at[s],
                        send_sem=psend.at[0], recv_sem=precv.at[s, j],
                        device_id=(0,),
                        device_id_type=pl.DeviceIdType.MESH).wait_recv()
            osum = po[...] + rxo[0] + rxo[1] + rxo[2]
            lsum = plb[...] + rxl[0] + rxl[1] + rxl[2]
            cg = (osum / lsum).astype(jnp.bfloat16)
            for h in range(HQ):
                ctx[0:GLOB, h * DH:(h + 1) * DH] = cg[h * GLOB:(h + 1) * GLOB, :]

        wob = wo_ref[...].astype(jnp.bfloat16)
        for c in range(SQ // CHUNK):
            out_ref[0, pl.ds(c * CHUNK, CHUNK), :] = lax.dot(
                ctx[pl.ds(c * CHUNK, CHUNK), :], wob,
                preferred_element_type=jnp.float32)

    return pl.pallas_call(
        body,
        out_shape=jax.ShapeDtypeStruct((1, SQ, DM), jnp.float32),
        in_specs=[
            pl.BlockSpec(memory_space=pl.ANY),
            pl.BlockSpec(memory_space=pltpu.VMEM),
            pl.BlockSpec(memory_space=pl.ANY),
            pl.BlockSpec(memory_space=pl.ANY),
            pl.BlockSpec(memory_space=pltpu.VMEM),
        ],
        out_specs=pl.BlockSpec(memory_space=pltpu.VMEM),
        scratch_shapes=[
            pltpu.VMEM((SQ, DM), jnp.bfloat16),
            pltpu.VMEM((KBUF, DM), jnp.bfloat16),
            pltpu.VMEM((KBUF, DM), jnp.bfloat16),
            pltpu.VMEM((GLOB, DM), jnp.bfloat16),
            pltpu.VMEM((SQ, DM), jnp.bfloat16),
            pltpu.VMEM((2, CHUNK, DM), jnp.float32),
            pltpu.VMEM((2, CHUNK, HQ, DH), jnp.float32),
            pltpu.VMEM((2, CHUNK, HQ, DH), jnp.float32),
            pltpu.VMEM((HQ * GLOB, DH), jnp.float32),
            pltpu.VMEM((HQ * GLOB, DH), jnp.float32),
            pltpu.VMEM((3, HQ * GLOB, DH), jnp.float32),
            pltpu.VMEM((3, HQ * GLOB, DH), jnp.float32),
            pltpu.SemaphoreType.DMA((2,)),
            pltpu.SemaphoreType.DMA((2,)),
            pltpu.SemaphoreType.DMA((2,)),
            pltpu.SemaphoreType.DMA((4,)),
            pltpu.SemaphoreType.DMA((4,)),
            pltpu.SemaphoreType.DMA((9,)),
            pltpu.SemaphoreType.DMA((3,)),
            pltpu.SemaphoreType.DMA((2,)),
            pltpu.SemaphoreType.DMA((3, 2)),
        ],
        compiler_params=pltpu.CompilerParams(
            collective_id=0, vmem_limit_bytes=47 * 1024 * 1024),
    )(x, Wq, K_ext, V_ext, Wo)


# device time: 64784 ns/iter; 2.6788x vs baseline; 1.0411x over previous
import jax
import jax.numpy as jnp
from jax import lax
from jax.experimental import pallas as pl
from jax.experimental.pallas import tpu as pltpu

N_DEV = 4
SQ = 2048
HQ = 8
DH = 128
DM = HQ * DH
HALO = 128
GLOB = 32
GSLOT = 128
OWN = GSLOT + HALO
KBUF = GSLOT + HALO + SQ + HALO
QBLK = 256
CHUNK = 512
SCALE = 0.08838834764831843
NEG = -1e9


def kernel(x, Wq, K_ext, V_ext, Wo):
    def body(x_ref, wq_ref, k_ref, v_ref, wo_ref, out_ref,
             qbuf, kbuf, vbuf, qg, ctx, xstage, kstage, vstage,
             po, plb, rxo, rxl,
             lsem, ksem, vsem, halo_send, halo_recv, gsend, grecv,
             psend, precv):
        my = lax.axis_index("i")
        left = lax.rem(my + N_DEV - 1, N_DEV)
        right = lax.rem(my + 1, N_DEV)

        bsem = pltpu.get_barrier_semaphore()
        for nbr in (left, right):
            pl.semaphore_signal(bsem, inc=1, device_id=(nbr,),
                                device_id_type=pl.DeviceIdType.MESH)
        on_diag = jnp.logical_or(my == 0, my == 2)

        @pl.when(on_diag)
        def _():
            other = jnp.where(my == 0, 2, 0)
            pl.semaphore_signal(bsem, inc=1, device_id=(other,),
                                device_id_type=pl.DeviceIdType.MESH)
            pl.semaphore_wait(bsem, 3)

        @pl.when(jnp.logical_not(on_diag))
        def _():
            pl.semaphore_wait(bsem, 2)

        wqb = wq_ref[...].astype(jnp.bfloat16)
        NC = SQ // CHUNK
        xc = [pltpu.make_async_copy(
            x_ref.at[0, pl.ds(c * CHUNK, CHUNK)], xstage.at[c % 2],
            lsem.at[c % 2]) for c in range(NC)]
        kc = [pltpu.make_async_copy(
            k_ref.at[0, pl.ds(c * CHUNK, CHUNK)], kstage.at[c % 2],
            ksem.at[c % 2]) for c in range(NC)]
        vc = [pltpu.make_async_copy(
            v_ref.at[0, pl.ds(c * CHUNK, CHUNK)], vstage.at[c % 2],
            vsem.at[c % 2]) for c in range(NC)]
        for d in (xc[0], xc[1], kc[0], kc[1], vc[0], vc[1]):
            d.start()
        for c in range(NC):
            xc[c].wait()
            qbuf[pl.ds(c * CHUNK, CHUNK), :] = (lax.dot(
                xstage[c % 2].astype(jnp.bfloat16), wqb,
                preferred_element_type=jnp.float32)
                * SCALE).astype(jnp.bfloat16)
            if c + 2 < NC:
                xc[c + 2].start()
            kc[c].wait()
            kbuf[pl.ds(OWN + c * CHUNK, CHUNK), :] = \
                kstage[c % 2].reshape(CHUNK, DM).astype(jnp.bfloat16)
            if c + 2 < NC:
                kc[c + 2].start()
            vc[c].wait()
            vbuf[pl.ds(OWN + c * CHUNK, CHUNK), :] = \
                vstage[c % 2].reshape(CHUNK, DM).astype(jnp.bfloat16)
            if c + 2 < NC:
                vc[c + 2].start()
        @pl.when(my == 0)
        def _():
            kbuf[0:GLOB, :] = kbuf[OWN:OWN + GLOB, :]
            vbuf[0:GLOB, :] = vbuf[OWN:OWN + GLOB, :]
            qg[...] = qbuf[0:GLOB, :]

        halo_rdmas = []
        for buf, s0 in ((kbuf, 0), (vbuf, 2)):
            halo_rdmas.append(pltpu.make_async_remote_copy(
                src_ref=buf.at[pl.ds(OWN, HALO)],
                dst_ref=buf.at[pl.ds(OWN + SQ, HALO)],
                send_sem=halo_send.at[s0], recv_sem=halo_recv.at[s0],
                device_id=(left,), device_id_type=pl.DeviceIdType.MESH))
            halo_rdmas.append(pltpu.make_async_remote_copy(
                src_ref=buf.at[pl.ds(OWN + SQ - HALO, HALO)],
                dst_ref=buf.at[pl.ds(GSLOT, HALO)],
                send_sem=halo_send.at[s0 + 1], recv_sem=halo_recv.at[s0 + 1],
                device_id=(right,), device_id_type=pl.DeviceIdType.MESH))
        for r in halo_rdmas:
            r.start()

        @pl.when(my == 0)
        def _():
            sends = []
            i = 0
            for dst in (1, 2, 3):
                for src_r, dst_r, j in (
                        (kbuf.at[pl.ds(OWN, GLOB)], kbuf.at[pl.ds(0, GLOB)], 0),
                        (vbuf.at[pl.ds(OWN, GLOB)], vbuf.at[pl.ds(0, GLOB)], 1),
                        (qbuf.at[pl.ds(0, GLOB)], qg, 2)):
                    d = pltpu.make_async_remote_copy(
                        src_ref=src_r, dst_ref=dst_r,
                        send_sem=gsend.at[i], recv_sem=grecv.at[j],
                        device_id=(dst,), device_id_type=pl.DeviceIdType.MESH)
                    d.start()
                    sends.append(d)
                    i += 1
            for d in sends:
                d.wait_send()

        def recv_only(dst_r, rsem):
            return pltpu.make_async_remote_copy(
                src_ref=dst_r, dst_ref=dst_r, send_sem=gsend.at[0],
                recv_sem=rsem, device_id=(0,),
                device_id_type=pl.DeviceIdType.MESH)

        @pl.when(my != 0)
        def _():
            recv_only(qg, grecv.at[2]).wait_recv()

        qgb = qg[...]
        for h in range(HQ):
            qh = qgb[:, h * DH:(h + 1) * DH]
            s = lax.dot_general(qh, kbuf[OWN:OWN + SQ, h * DH:(h + 1) * DH],
                                (((1,), (1,)), ((), ())),
                                preferred_element_type=jnp.float32)
            w = jnp.exp(s)
            l = jnp.sum(w, axis=1, keepdims=True)
            o = lax.dot_general(w.astype(jnp.bfloat16),
                                vbuf[OWN:OWN + SQ, h * DH:(h + 1) * DH],
                                (((1,), (0,)), ((), ())),
                                preferred_element_type=jnp.float32)
            po[h * GLOB:(h + 1) * GLOB, :] = o
            plb[h * GLOB:(h + 1) * GLOB, :] = jnp.broadcast_to(l, (GLOB, DH))

        for src in (1, 2, 3):
            @pl.when(my == src)
            def _(src=src):
                ds = []
                for j, (sbuf, rbuf) in enumerate(((po, rxo), (plb, rxl))):
                    d = pltpu.make_async_remote_copy(
                        src_ref=sbuf, dst_ref=rbuf.at[src - 1],
                        send_sem=psend.at[j], recv_sem=precv.at[src - 1, j],
                        device_id=(0,), device_id_type=pl.DeviceIdType.MESH)
                    d.start()
                    ds.append(d)
                for d in ds:
                    d.wait_send()

        for r in halo_rdmas:
            r.wait()

        @pl.when(my != 0)
        def _():
            recv_only(kbuf.at[pl.ds(0, GLOB)], grecv.at[0]).wait_recv()
            recv_only(vbuf.at[pl.ds(0, GLOB)], grecv.at[1]).wait_recv()

        WWIN = QBLK + 2 * HALO

        for qb in range(SQ // QBLK):
            q0 = qb * QBLK
            cw = lax.broadcasted_iota(jnp.int32, (1, WWIN), 1)
            ciw = GSLOT + q0 + cw
            kiw = my * SQ + ciw - OWN
            qi = my * SQ + q0 + lax.broadcasted_iota(jnp.int32, (QBLK, 1), 0)
            band = (kiw >= qi - HALO) & (kiw <= qi + HALO) & (kiw >= GLOB)
            band = band & jnp.logical_not(
                jnp.logical_and(my == N_DEV - 1, ciw >= OWN + SQ))
            bias = jnp.where(band, 0.0, NEG).astype(jnp.float32)

            for h in range(HQ):
                c0 = h * DH
                qh = qbuf[pl.ds(q0, QBLK), pl.ds(c0, DH)]
                sw = lax.dot_general(qh, kbuf[pl.ds(GSLOT + q0, WWIN),
                                              pl.ds(c0, DH)],
                                     (((1,), (1,)), ((), ())),
                                     preferred_element_type=jnp.float32)
                ww = jnp.exp(sw + bias)
                sg = lax.dot_general(qh, kbuf[0:GLOB, pl.ds(c0, DH)],
                                     (((1,), (1,)), ((), ())),
                                     preferred_element_type=jnp.float32)
                wg = jnp.exp(sg)
                l = (jnp.sum(ww, axis=1, keepdims=True)
                     + jnp.sum(wg, axis=1, keepdims=True))
                o = lax.dot_general(ww.astype(jnp.bfloat16),
                                    vbuf[pl.ds(GSLOT + q0, WWIN),
                                         pl.ds(c0, DH)],
                                    (((1,), (0,)), ((), ())),
                                    preferred_element_type=jnp.float32)
                o = o + lax.dot_general(wg.astype(jnp.bfloat16),
                                        vbuf[0:GLOB, pl.ds(c0, DH)],
                                        (((1,), (0,)), ((), ())),
                                        preferred_element_type=jnp.float32)
                ctx[pl.ds(q0, QBLK), pl.ds(c0, DH)] = \
                    (o / l).astype(jnp.bfloat16)

        @pl.when(my == 0)
        def _():
            for s in range(3):
                for j, rbuf in enumerate((rxo, rxl)):
                    pltpu.make_async_remote_copy(
                        src_ref=rbuf.at[s], dst_ref=rbuf.at[s],
                        send_sem=psend.at[0], recv_sem=precv.at[s, j],
                        device_id=(0,),
                        device_id_type=pl.DeviceIdType.MESH).wait_recv()
            osum = po[...] + rxo[0] + rxo[1] + rxo[2]
            lsum = plb[...] + rxl[0] + rxl[1] + rxl[2]
            cg = (osum / lsum).astype(jnp.bfloat16)
            for h in range(HQ):
                ctx[0:GLOB, h * DH:(h + 1) * DH] = cg[h * GLOB:(h + 1) * GLOB, :]

        wob = wo_ref[...].astype(jnp.bfloat16)
        for c in range(SQ // CHUNK):
            out_ref[0, pl.ds(c * CHUNK, CHUNK), :] = lax.dot(
                ctx[pl.ds(c * CHUNK, CHUNK), :], wob,
                preferred_element_type=jnp.float32)

    return pl.pallas_call(
        body,
        out_shape=jax.ShapeDtypeStruct((1, SQ, DM), jnp.float32),
        in_specs=[
            pl.BlockSpec(memory_space=pl.ANY),
            pl.BlockSpec(memory_space=pltpu.VMEM),
            pl.BlockSpec(memory_space=pl.ANY),
            pl.BlockSpec(memory_space=pl.ANY),
            pl.BlockSpec(memory_space=pltpu.VMEM),
        ],
        out_specs=pl.BlockSpec(memory_space=pltpu.VMEM),
        scratch_shapes=[
            pltpu.VMEM((SQ, DM), jnp.bfloat16),
            pltpu.VMEM((KBUF, DM), jnp.bfloat16),
            pltpu.VMEM((KBUF, DM), jnp.bfloat16),
            pltpu.VMEM((GLOB, DM), jnp.bfloat16),
            pltpu.VMEM((SQ, DM), jnp.bfloat16),
            pltpu.VMEM((2, CHUNK, DM), jnp.float32),
            pltpu.VMEM((2, CHUNK, HQ, DH), jnp.float32),
            pltpu.VMEM((2, CHUNK, HQ, DH), jnp.float32),
            pltpu.VMEM((HQ * GLOB, DH), jnp.float32),
            pltpu.VMEM((HQ * GLOB, DH), jnp.float32),
            pltpu.VMEM((3, HQ * GLOB, DH), jnp.float32),
            pltpu.VMEM((3, HQ * GLOB, DH), jnp.float32),
            pltpu.SemaphoreType.DMA((2,)),
            pltpu.SemaphoreType.DMA((2,)),
            pltpu.SemaphoreType.DMA((2,)),
            pltpu.SemaphoreType.DMA((4,)),
            pltpu.SemaphoreType.DMA((4,)),
            pltpu.SemaphoreType.DMA((9,)),
            pltpu.SemaphoreType.DMA((3,)),
            pltpu.SemaphoreType.DMA((2,)),
            pltpu.SemaphoreType.DMA((3, 2)),
        ],
        compiler_params=pltpu.CompilerParams(
            collective_id=0, vmem_limit_bytes=47 * 1024 * 1024),
    )(x, Wq, K_ext, V_ext, Wo)
